# Initial kernel scaffold; baseline (speedup 1.0000x reference)
import jax
import jax.numpy as jnp
from jax.experimental import pallas as pl


def kernel(x, edge_index, W1, b1, W_mu, b_mu, W_lv, b_lv):
    raise NotImplementedError("write your pallas kernel here")


# speedup vs baseline: 9.8875x; 9.8875x over previous
import functools

import jax
import jax.numpy as jnp
from jax import lax
from jax.experimental import pallas as pl
from jax.experimental.pallas import tpu as pltpu
from jax.experimental.pallas import tpu_sc as plsc

N = 10000
E = 320000
D = 128
DH = 64
NC, NS = 2, 16
NW = NC * NS
GP = 128
NP = 10240
EG = 2560
GPW = EG // NW
RPT = NP // NS
DEG_W = 16
TBLK = 2048

_mesh = plsc.VectorSubcoreMesh(
    core_axis_name="c", subcore_axis_name="s", num_cores=NC, num_subcores=NS
)
_sc_params = pltpu.CompilerParams(use_tc_tiling_on_sc=False)


def _worker_ids():
    c = lax.axis_index("c")
    s = lax.axis_index("s")
    return c, s, c * NS + s


def _fill_zeros(zbuf):
    zeros = jnp.zeros((16,), jnp.float32)
    nvec = zbuf.shape[1] // 16

    def fill(i, _):
        zbuf[i // nvec, pl.ds((i % nvec) * 16, 16)] = zeros
        return 0

    lax.fori_loop(0, 128 * nvec, fill, 0)


def _zero_my_slice(zbuf, acc, s):
    for j in range(RPT // 128):
        pltpu.sync_copy(zbuf, acc.at[pl.ds(s * RPT + j * 128, 128)])


@functools.partial(
    pl.kernel,
    out_type=jax.ShapeDtypeStruct((NC, NP, DEG_W), jnp.float32),
    mesh=_mesh,
    scratch_types=[
        pltpu.VMEM((GPW, GP), jnp.int32),
        pltpu.VMEM((GP, DEG_W), jnp.float32),
        pltpu.VMEM((128, DEG_W), jnp.float32),
        pltpu.VMEM_SHARED((NP, DEG_W), jnp.float32),
    ],
    compiler_params=_sc_params,
)
def _deg_kernel(dst_hbm, out_hbm, idx_d, ones_v, zbuf, acc):
    c, s, w = _worker_ids()
    one = jnp.ones((16,), jnp.float32)

    def fill_ones(i, _):
        ones_v[i, pl.ds(0, 16)] = one
        return 0

    lax.fori_loop(0, GP, fill_ones, 0)
    _fill_zeros(zbuf)
    _zero_my_slice(zbuf, acc, s)
    pltpu.sync_copy(dst_hbm.at[pl.ds(w * GPW, GPW)], idx_d)
    plsc.subcore_barrier()

    def step(k, _):
        pltpu.sync_copy(ones_v, acc.at[idx_d.at[k]], add=True)
        return 0

    lax.fori_loop(0, GPW, step, 0)
    plsc.subcore_barrier()
    pltpu.sync_copy(
        acc.at[pl.ds(s * RPT, RPT)], out_hbm.at[c, pl.ds(s * RPT, RPT)]
    )


@functools.partial(
    pl.kernel,
    out_type=jax.ShapeDtypeStruct((NC, 2, NP, DH), jnp.float32),
    mesh=_mesh,
    scratch_types=[
        pltpu.VMEM((GPW, GP), jnp.int32),
        pltpu.VMEM((GPW, GP), jnp.int32),
        pltpu.VMEM((GP, DH), jnp.float32),
        pltpu.VMEM((128, DH), jnp.float32),
        pltpu.VMEM_SHARED((NP, DH), jnp.float32),
    ],
    compiler_params=_sc_params,
)
def _agg_kernel(ga_hbm, gb_hbm, src_hbm, dst_hbm, out_hbm,
                idx_s, idx_d, rows, zbuf, acc):
    c, s, w = _worker_ids()
    _fill_zeros(zbuf)
    pltpu.sync_copy(src_hbm.at[pl.ds(w * GPW, GPW)], idx_s)
    pltpu.sync_copy(dst_hbm.at[pl.ds(w * GPW, GPW)], idx_d)
    for half, tbl in ((0, ga_hbm), (1, gb_hbm)):
        _zero_my_slice(zbuf, acc, s)
        plsc.subcore_barrier()

        def step(k, _):
            pltpu.sync_copy(tbl.at[idx_s.at[k]], rows)
            pltpu.sync_copy(rows, acc.at[idx_d.at[k]], add=True)
            return 0

        lax.fori_loop(0, GPW, step, 0)
        plsc.subcore_barrier()
        pltpu.sync_copy(
            acc.at[pl.ds(s * RPT, RPT)],
            out_hbm.at[c, half, pl.ds(s * RPT, RPT)],
        )
        plsc.subcore_barrier()


def _tc1_body(x_ref, w1_ref, degp_ref, ga_ref, gb_ref, dinv_ref):
    deg = degp_ref[0][:, 0:1] + degp_ref[1][:, 0:1] + 1.0
    dinv = lax.rsqrt(deg)
    h = jnp.dot(x_ref[...], w1_ref[...], preferred_element_type=jnp.float32)
    g = h * dinv
    ga_ref[...] = g[:, :DH]
    gb_ref[...] = g[:, DH:]
    dinv_ref[...] = dinv


def _tc2_body(s1_ref, ga_ref, gb_ref, dinv_ref, b1_ref, wcat_ref,
              ga2_ref, gb2_ref):
    dinv = dinv_ref[...]
    sa = s1_ref[0, 0] + s1_ref[1, 0] + ga_ref[...]
    sb = s1_ref[0, 1] + s1_ref[1, 1] + gb_ref[...]
    pre = jnp.concatenate([sa, sb], axis=1) * dinv + b1_ref[...]
    h = jnp.maximum(pre, 0.0)
    g2 = jnp.dot(h, wcat_ref[...], preferred_element_type=jnp.float32) * dinv
    ga2_ref[...] = g2[:, :DH]
    gb2_ref[...] = g2[:, DH:]


def _tc3_body(s2_ref, ga_ref, gb_ref, dinv_ref, bcat_ref, out_ref):
    sa = s2_ref[0, 0] + s2_ref[1, 0] + ga_ref[...]
    sb = s2_ref[0, 1] + s2_ref[1, 1] + gb_ref[...]
    out_ref[...] = (
        jnp.concatenate([sa, sb], axis=1) * dinv_ref[...] + bcat_ref[...]
    )


_row_spec = pl.BlockSpec((TBLK, D), lambda i: (i, 0))
_half_spec = pl.BlockSpec((TBLK, DH), lambda i: (i, 0))
_col_spec = pl.BlockSpec((TBLK, 1), lambda i: (i, 0))
_par_spec = pl.BlockSpec((NC, 2, TBLK, DH), lambda i: (0, 0, i, 0))
_w_spec = pl.BlockSpec((D, D), lambda i: (0, 0))
_b_spec = pl.BlockSpec((1, D), lambda i: (0, 0))
_GRID = (NP // TBLK,)

_half_sds = jax.ShapeDtypeStruct((NP, DH), jnp.float32)


def _tc1(xp, W1, degp):
    return pl.pallas_call(
        _tc1_body,
        grid=_GRID,
        in_specs=[
            _row_spec,
            _w_spec,
            pl.BlockSpec((NC, TBLK, DEG_W), lambda i: (0, i, 0)),
        ],
        out_specs=[_half_spec, _half_spec, _col_spec],
        out_shape=[_half_sds, _half_sds,
                   jax.ShapeDtypeStruct((NP, 1), jnp.float32)],
    )(xp, W1, degp)


def _tc2(s1, ga, gb, dinv, b1, wcat):
    return pl.pallas_call(
        _tc2_body,
        grid=_GRID,
        in_specs=[_par_spec, _half_spec, _half_spec, _col_spec, _b_spec,
                  _w_spec],
        out_specs=[_half_spec, _half_spec],
        out_shape=[_half_sds, _half_sds],
    )(s1, ga, gb, dinv, b1, wcat)


def _tc3(s2, ga, gb, dinv, bcat):
    return pl.pallas_call(
        _tc3_body,
        grid=_GRID,
        in_specs=[_par_spec, _half_spec, _half_spec, _col_spec, _b_spec],
        out_specs=_row_spec,
        out_shape=jax.ShapeDtypeStruct((NP, D), jnp.float32),
    )(s2, ga, gb, dinv, bcat)


def kernel(x, edge_index, W1, b1, W_mu, b_mu, W_lv, b_lv):
    pad_e = EG * GP - E
    src = jnp.concatenate(
        [edge_index[0], jnp.full((pad_e,), N, jnp.int32)]
    ).reshape(EG, GP)
    dst = jnp.concatenate(
        [edge_index[1], jnp.full((pad_e,), N, jnp.int32)]
    ).reshape(EG, GP)
    xp = jnp.zeros((NP, D), jnp.float32).at[:N].set(x)
    wcat = jnp.concatenate([W_mu, W_lv], axis=1)
    bcat = jnp.concatenate([b_mu, b_lv]).reshape(1, D)

    degp = _deg_kernel(dst)
    ga1, gb1, dinv = _tc1(xp, W1, degp)
    s1 = _agg_kernel(ga1, gb1, src, dst)
    ga2, gb2 = _tc2(s1, ga1, gb1, dinv, b1.reshape(1, D), wcat)
    s2 = _agg_kernel(ga2, gb2, src, dst)
    out = _tc3(s2, ga2, gb2, dinv, bcat)
    return out[:N, :64], out[:N, 64:]


# speedup vs baseline: 10.5976x; 1.0718x over previous
import functools

import jax
import jax.numpy as jnp
from jax import lax
from jax.experimental import pallas as pl
from jax.experimental.pallas import tpu as pltpu
from jax.experimental.pallas import tpu_sc as plsc

N = 10000
E = 320000
D = 128
DH = 64
DQ = 32
NC, NS = 2, 16
NW = NC * NS
GP = 128
NP = 10240
EG = 2560
GPW = EG // NW
RPT = NP // NS
DEG_W = 16
TBLK = 2048

_mesh = plsc.VectorSubcoreMesh(
    core_axis_name="c", subcore_axis_name="s", num_cores=NC, num_subcores=NS
)
_sc_params = pltpu.CompilerParams(use_tc_tiling_on_sc=False)


def _worker_ids():
    c = lax.axis_index("c")
    s = lax.axis_index("s")
    return c, s, c * NS + s


def _fill_zeros(zbuf):
    zeros = jnp.zeros((16,), jnp.float32)
    nvec = zbuf.shape[1] // 16

    def fill(i, _):
        zbuf[i // nvec, pl.ds((i % nvec) * 16, 16)] = zeros
        return 0

    lax.fori_loop(0, 128 * nvec, fill, 0)


def _zero_my_slice(zbuf, acc, s):
    for j in range(RPT // 128):
        pltpu.sync_copy(zbuf, acc.at[pl.ds(s * RPT + j * 128, 128)])


@functools.partial(
    pl.kernel,
    out_type=jax.ShapeDtypeStruct((NC, NP, DEG_W), jnp.float32),
    mesh=_mesh,
    scratch_types=[
        pltpu.VMEM((GPW, GP), jnp.int32),
        pltpu.VMEM((GP, DEG_W), jnp.float32),
        pltpu.VMEM((128, DEG_W), jnp.float32),
        pltpu.VMEM_SHARED((NP, DEG_W), jnp.float32),
    ],
    compiler_params=_sc_params,
)
def _deg_kernel(dst_hbm, out_hbm, idx_d, ones_v, zbuf, acc):
    c, s, w = _worker_ids()
    one = jnp.ones((16,), jnp.float32)

    def fill_ones(i, _):
        ones_v[i, pl.ds(0, 16)] = one
        return 0

    lax.fori_loop(0, GP, fill_ones, 0)
    _fill_zeros(zbuf)
    _zero_my_slice(zbuf, acc, s)
    pltpu.sync_copy(dst_hbm.at[pl.ds(w * GPW, GPW)], idx_d)
    plsc.subcore_barrier()

    def step(k, _):
        pltpu.sync_copy(ones_v, acc.at[idx_d.at[k]], add=True)
        return 0

    lax.fori_loop(0, GPW, step, 0)
    plsc.subcore_barrier()
    pltpu.sync_copy(
        acc.at[pl.ds(s * RPT, RPT)], out_hbm.at[c, pl.ds(s * RPT, RPT)]
    )


NBUF = 8
PD = NBUF // 2


def _make_agg(width, nsweeps):
    @functools.partial(
        pl.kernel,
        out_type=jax.ShapeDtypeStruct((NC, nsweeps, NP, width), jnp.float32),
        mesh=_mesh,
        scratch_types=[
            pltpu.VMEM((GPW, GP), jnp.int32),
            pltpu.VMEM((GPW, GP), jnp.int32),
            pltpu.VMEM((NBUF, GP, width), jnp.float32),
            pltpu.VMEM((128, width), jnp.float32),
            pltpu.VMEM_SHARED((NP, width), jnp.float32),
            pltpu.SemaphoreType.DMA((NBUF,)),
            pltpu.SemaphoreType.DMA((NBUF,)),
        ],
        compiler_params=_sc_params,
    )
    def agg(*refs):
        tbls = refs[:nsweeps]
        src_hbm, dst_hbm, out_hbm = refs[nsweeps:nsweeps + 3]
        idx_s, idx_d, rows, zbuf, acc, gsem, ssem = refs[nsweeps + 3:]
        c, s, w = _worker_ids()
        _fill_zeros(zbuf)
        pltpu.sync_copy(src_hbm.at[pl.ds(w * GPW, GPW)], idx_s)
        pltpu.sync_copy(dst_hbm.at[pl.ds(w * GPW, GPW)], idx_d)

        for sweep, tbl in enumerate(tbls):
            _zero_my_slice(zbuf, acc, s)
            plsc.subcore_barrier()

            def start_gather(k, b):
                pltpu.async_copy(tbl.at[idx_s.at[k]], rows.at[b], gsem.at[b])

            def wait_gather(b):
                pltpu.make_async_copy(
                    tbl.at[idx_s.at[0]], rows.at[b], gsem.at[b]
                ).wait()

            def start_scatter(k, b):
                pltpu.async_copy(rows.at[b], acc.at[idx_d.at[k]], ssem.at[b],
                                 add=True)

            def wait_scatter(b):
                pltpu.make_async_copy(
                    rows.at[b], acc.at[idx_d.at[0]], ssem.at[b]
                ).wait()

            for b in range(PD):
                start_gather(b, b)

            def visit(k, b):
                wait_gather(b)
                start_scatter(k, b)

                @pl.when(k >= PD)
                def _():
                    wait_scatter((b + PD) % NBUF)

                @pl.when(k + PD < GPW)
                def _():
                    start_gather(k + PD, (b + PD) % NBUF)

            def macro(j, _):
                for b in range(NBUF):
                    visit(j * NBUF + b, b)
                return 0

            lax.fori_loop(0, GPW // NBUF, macro, 0)
            for b in range(PD):
                wait_scatter((GPW - PD + b) % NBUF)
            plsc.subcore_barrier()
            pltpu.sync_copy(
                acc.at[pl.ds(s * RPT, RPT)],
                out_hbm.at[c, sweep, pl.ds(s * RPT, RPT)],
            )
            plsc.subcore_barrier()

    return agg


_agg1 = _make_agg(DQ, 4)
_agg2 = _make_agg(DQ, 4)


def _tc1_body(x_ref, w1_ref, degp_ref,
              gq0_ref, gq1_ref, gq2_ref, gq3_ref, dinv_ref):
    deg = degp_ref[0][:, 0:1] + degp_ref[1][:, 0:1] + 1.0
    dinv = lax.rsqrt(deg)
    h = jnp.dot(x_ref[...], w1_ref[...], preferred_element_type=jnp.float32)
    g = h * dinv
    for q, ref in enumerate((gq0_ref, gq1_ref, gq2_ref, gq3_ref)):
        ref[...] = g[:, q * DQ:(q + 1) * DQ]
    dinv_ref[...] = dinv


def _tc2_body(s1_ref, gq0_ref, gq1_ref, gq2_ref, gq3_ref,
              dinv_ref, b1_ref, wcat_ref,
              g2q0_ref, g2q1_ref, g2q2_ref, g2q3_ref):
    dinv = dinv_ref[...]
    gq = (gq0_ref, gq1_ref, gq2_ref, gq3_ref)
    parts = [s1_ref[0, q] + s1_ref[1, q] + gq[q][...] for q in range(4)]
    pre = jnp.concatenate(parts, axis=1) * dinv + b1_ref[...]
    h = jnp.maximum(pre, 0.0)
    g2 = jnp.dot(h, wcat_ref[...], preferred_element_type=jnp.float32) * dinv
    for q, ref in enumerate((g2q0_ref, g2q1_ref, g2q2_ref, g2q3_ref)):
        ref[...] = g2[:, q * DQ:(q + 1) * DQ]


def _tc3_body(s2_ref, g2q0_ref, g2q1_ref, g2q2_ref, g2q3_ref,
              dinv_ref, bcat_ref, out_ref):
    gq = (g2q0_ref, g2q1_ref, g2q2_ref, g2q3_ref)
    parts = [s2_ref[0, q] + s2_ref[1, q] + gq[q][...] for q in range(4)]
    out_ref[...] = (
        jnp.concatenate(parts, axis=1) * dinv_ref[...] + bcat_ref[...]
    )


_row_spec = pl.BlockSpec((TBLK, D), lambda i: (i, 0))
_quart_spec = pl.BlockSpec((TBLK, DQ), lambda i: (i, 0))
_col_spec = pl.BlockSpec((TBLK, 1), lambda i: (i, 0))
_par_spec = pl.BlockSpec((NC, 4, TBLK, DQ), lambda i: (0, 0, i, 0))
_w_spec = pl.BlockSpec((D, D), lambda i: (0, 0))
_b_spec = pl.BlockSpec((1, D), lambda i: (0, 0))
_GRID = (NP // TBLK,)

_quart_sds = jax.ShapeDtypeStruct((NP, DQ), jnp.float32)


def _tc1(xp, W1, degp):
    return pl.pallas_call(
        _tc1_body,
        grid=_GRID,
        in_specs=[
            _row_spec,
            _w_spec,
            pl.BlockSpec((NC, TBLK, DEG_W), lambda i: (0, i, 0)),
        ],
        out_specs=[_quart_spec] * 4 + [_col_spec],
        out_shape=[_quart_sds] * 4 +
                  [jax.ShapeDtypeStruct((NP, 1), jnp.float32)],
    )(xp, W1, degp)


def _tc2(s1, g1q, dinv, b1, wcat):
    return pl.pallas_call(
        _tc2_body,
        grid=_GRID,
        in_specs=[_par_spec] + [_quart_spec] * 4 + [_col_spec, _b_spec,
                  _w_spec],
        out_specs=[_quart_spec] * 4,
        out_shape=[_quart_sds] * 4,
    )(s1, *g1q, dinv, b1, wcat)


def _tc3(s2, g2q, dinv, bcat):
    return pl.pallas_call(
        _tc3_body,
        grid=_GRID,
        in_specs=[_par_spec] + [_quart_spec] * 4 + [_col_spec, _b_spec],
        out_specs=_row_spec,
        out_shape=jax.ShapeDtypeStruct((NP, D), jnp.float32),
    )(s2, *g2q, dinv, bcat)


def kernel(x, edge_index, W1, b1, W_mu, b_mu, W_lv, b_lv):
    pad_e = EG * GP - E
    src = jnp.concatenate(
        [edge_index[0], jnp.full((pad_e,), N, jnp.int32)]
    ).reshape(EG, GP)
    dst = jnp.concatenate(
        [edge_index[1], jnp.full((pad_e,), N, jnp.int32)]
    ).reshape(EG, GP)
    xp = jnp.zeros((NP, D), jnp.float32).at[:N].set(x)
    wcat = jnp.concatenate([W_mu, W_lv], axis=1)
    bcat = jnp.concatenate([b_mu, b_lv]).reshape(1, D)

    degp = _deg_kernel(dst)
    *g1q, dinv = _tc1(xp, W1, degp)
    s1 = _agg1(*g1q, src, dst)
    g2q = _tc2(s1, g1q, dinv, b1.reshape(1, D), wcat)
    s2 = _agg2(*g2q, src, dst)
    out = _tc3(s2, g2q, dinv, bcat)
    return out[:N, :64], out[:N, 64:]


# speedup vs baseline: 25.7690x; 2.4316x over previous
import functools

import jax
import jax.numpy as jnp
from jax import lax
from jax.experimental import pallas as pl
from jax.experimental.pallas import tpu as pltpu
from jax.experimental.pallas import tpu_sc as plsc

N = 10000
E = 320000
D = 128
DH = 64
DQ = 32
NC, NS = 2, 16
NW = NC * NS
GP = 128
NP = 10240
EG = 2560
GPW = EG // NW
RPT = NP // NS
DEG_W = 16
TBLK = 2048

_mesh = plsc.VectorSubcoreMesh(
    core_axis_name="c", subcore_axis_name="s", num_cores=NC, num_subcores=NS
)
_sc_params = pltpu.CompilerParams(use_tc_tiling_on_sc=False)


def _worker_ids():
    c = lax.axis_index("c")
    s = lax.axis_index("s")
    return c, s, c * NS + s


def _fill_zeros(zbuf):
    zeros = jnp.zeros((16,), jnp.float32)
    nvec = zbuf.shape[1] // 16

    def fill(i, _):
        zbuf[i // nvec, pl.ds((i % nvec) * 16, 16)] = zeros
        return 0

    lax.fori_loop(0, 128 * nvec, fill, 0)


def _zero_my_slice(zbuf, acc, s):
    for j in range(RPT // 128):
        pltpu.sync_copy(zbuf, acc.at[pl.ds(s * RPT + j * 128, 128)])


@functools.partial(
    pl.kernel,
    out_type=jax.ShapeDtypeStruct((NC, NP, DEG_W), jnp.float32),
    mesh=_mesh,
    scratch_types=[
        pltpu.VMEM((GPW, GP), jnp.int32),
        pltpu.VMEM((GP, DEG_W), jnp.float32),
        pltpu.VMEM((128, DEG_W), jnp.float32),
        pltpu.VMEM_SHARED((NP, DEG_W), jnp.float32),
    ],
    compiler_params=_sc_params,
)
def _deg_kernel(dst_hbm, out_hbm, idx_d, ones_v, zbuf, acc):
    c, s, w = _worker_ids()
    one = jnp.ones((16,), jnp.float32)

    def fill_ones(i, _):
        ones_v[i, pl.ds(0, 16)] = one
        return 0

    lax.fori_loop(0, GP, fill_ones, 0)
    _fill_zeros(zbuf)
    _zero_my_slice(zbuf, acc, s)
    pltpu.sync_copy(dst_hbm.at[pl.ds(w * GPW, GPW)], idx_d)
    plsc.subcore_barrier()

    def step(k, _):
        pltpu.sync_copy(ones_v, acc.at[idx_d.at[k]], add=True)
        return 0

    lax.fori_loop(0, GPW, step, 0)
    plsc.subcore_barrier()
    pltpu.sync_copy(
        acc.at[pl.ds(s * RPT, RPT)], out_hbm.at[c, pl.ds(s * RPT, RPT)]
    )


NBUF = 8
PD = NBUF // 2


def _make_agg(width, nsweeps):
    @functools.partial(
        pl.kernel,
        out_type=jax.ShapeDtypeStruct((NC, nsweeps, NP, width), jnp.float32),
        mesh=_mesh,
        scratch_types=[
            pltpu.VMEM((GPW, GP), jnp.int32),
            pltpu.VMEM((GPW, GP), jnp.int32),
            pltpu.VMEM((NBUF, GP, width), jnp.float32),
            pltpu.VMEM((128, width), jnp.float32),
            pltpu.VMEM_SHARED((NP, width), jnp.float32),
            pltpu.SemaphoreType.DMA((NBUF,)),
            pltpu.SemaphoreType.DMA((NBUF,)),
        ],
        compiler_params=_sc_params,
    )
    def agg(*refs):
        tbls = refs[:nsweeps]
        src_hbm, dst_hbm, out_hbm = refs[nsweeps:nsweeps + 3]
        idx_s, idx_d, rows, zbuf, acc, gsem, ssem = refs[nsweeps + 3:]
        c, s, w = _worker_ids()
        _fill_zeros(zbuf)
        pltpu.sync_copy(src_hbm.at[pl.ds(w * GPW, GPW)], idx_s)
        pltpu.sync_copy(dst_hbm.at[pl.ds(w * GPW, GPW)], idx_d)

        for sweep, tbl in enumerate(tbls):
            _zero_my_slice(zbuf, acc, s)
            plsc.subcore_barrier()

            def start_gather(k, b):
                pltpu.async_copy(tbl.at[idx_s.at[k]], rows.at[b], gsem.at[b])

            def wait_gather(b):
                pltpu.make_async_copy(
                    tbl.at[idx_s.at[0]], rows.at[b], gsem.at[b]
                ).wait()

            def start_scatter(k, b):
                pltpu.async_copy(rows.at[b], acc.at[idx_d.at[k]], ssem.at[b],
                                 add=True)

            def wait_scatter(b):
                pltpu.make_async_copy(
                    rows.at[b], acc.at[idx_d.at[0]], ssem.at[b]
                ).wait()

            for b in range(PD):
                start_gather(b, b)

            def visit(k, b):
                wait_gather(b)
                start_scatter(k, b)

                @pl.when(k >= PD)
                def _():
                    wait_scatter((b + PD) % NBUF)

                @pl.when(k + PD < GPW)
                def _():
                    start_gather(k + PD, (b + PD) % NBUF)

            def macro(j, _):
                for b in range(NBUF):
                    visit(j * NBUF + b, b)
                return 0

            lax.fori_loop(0, GPW // NBUF, macro, 0)
            for b in range(PD):
                wait_scatter((GPW - PD + b) % NBUF)
            plsc.subcore_barrier()
            pltpu.sync_copy(
                acc.at[pl.ds(s * RPT, RPT)],
                out_hbm.at[c, sweep, pl.ds(s * RPT, RPT)],
            )
            plsc.subcore_barrier()

    return agg


_agg1 = _make_agg(DQ, 4)
_agg2 = _make_agg(DQ, 4)


def _tc1_body(x_ref, w1_ref, degp_ref,
              gq0_ref, gq1_ref, gq2_ref, gq3_ref, dinv_ref):
    deg = degp_ref[0][:, 0:1] + degp_ref[1][:, 0:1] + 1.0
    dinv = lax.rsqrt(deg)
    h = jnp.dot(x_ref[...], w1_ref[...], preferred_element_type=jnp.float32)
    g = h * dinv
    for q, ref in enumerate((gq0_ref, gq1_ref, gq2_ref, gq3_ref)):
        ref[...] = g[:, q * DQ:(q + 1) * DQ]
    dinv_ref[...] = dinv


def _tc2_body(s1_ref, gq0_ref, gq1_ref, gq2_ref, gq3_ref,
              dinv_ref, b1_ref, wcat_ref,
              g2q0_ref, g2q1_ref, g2q2_ref, g2q3_ref):
    dinv = dinv_ref[...]
    gq = (gq0_ref, gq1_ref, gq2_ref, gq3_ref)
    parts = [s1_ref[0, q] + s1_ref[1, q] + gq[q][...] for q in range(4)]
    pre = jnp.concatenate(parts, axis=1) * dinv + b1_ref[...]
    h = jnp.maximum(pre, 0.0)
    g2 = jnp.dot(h, wcat_ref[...], preferred_element_type=jnp.float32) * dinv
    for q, ref in enumerate((g2q0_ref, g2q1_ref, g2q2_ref, g2q3_ref)):
        ref[...] = g2[:, q * DQ:(q + 1) * DQ]


def _tc3_body(s2_ref, g2q0_ref, g2q1_ref, g2q2_ref, g2q3_ref,
              dinv_ref, bcat_ref, out_ref):
    gq = (g2q0_ref, g2q1_ref, g2q2_ref, g2q3_ref)
    parts = [s2_ref[0, q] + s2_ref[1, q] + gq[q][...] for q in range(4)]
    out_ref[...] = (
        jnp.concatenate(parts, axis=1) * dinv_ref[...] + bcat_ref[...]
    )


_row_spec = pl.BlockSpec((TBLK, D), lambda i: (i, 0))
_quart_spec = pl.BlockSpec((TBLK, DQ), lambda i: (i, 0))
_col_spec = pl.BlockSpec((TBLK, 1), lambda i: (i, 0))
_par_spec = pl.BlockSpec((NC, 4, TBLK, DQ), lambda i: (0, 0, i, 0))
_w_spec = pl.BlockSpec((D, D), lambda i: (0, 0))
_b_spec = pl.BlockSpec((1, D), lambda i: (0, 0))
_GRID = (NP // TBLK,)

_quart_sds = jax.ShapeDtypeStruct((NP, DQ), jnp.float32)


def _tc1(xp, W1, degp):
    return pl.pallas_call(
        _tc1_body,
        grid=_GRID,
        in_specs=[
            _row_spec,
            _w_spec,
            pl.BlockSpec((NC, TBLK, DEG_W), lambda i: (0, i, 0)),
        ],
        out_specs=[_quart_spec] * 4 + [_col_spec],
        out_shape=[_quart_sds] * 4 +
                  [jax.ShapeDtypeStruct((NP, 1), jnp.float32)],
    )(xp, W1, degp)


def _tc2(s1, g1q, dinv, b1, wcat):
    return pl.pallas_call(
        _tc2_body,
        grid=_GRID,
        in_specs=[_par_spec] + [_quart_spec] * 4 + [_col_spec, _b_spec,
                  _w_spec],
        out_specs=[_quart_spec] * 4,
        out_shape=[_quart_sds] * 4,
    )(s1, *g1q, dinv, b1, wcat)


def _tc3(s2, g2q, dinv, bcat):
    return pl.pallas_call(
        _tc3_body,
        grid=_GRID,
        in_specs=[_par_spec] + [_quart_spec] * 4 + [_col_spec, _b_spec],
        out_specs=_row_spec,
        out_shape=jax.ShapeDtypeStruct((NP, D), jnp.float32),
    )(s2, *g2q, dinv, bcat)


def kernel(x, edge_index, W1, b1, W_mu, b_mu, W_lv, b_lv):
    pad_e = EG * GP - E
    pad_idx = N + jnp.arange(pad_e, dtype=jnp.int32) % (NP - N)
    src = jnp.concatenate([edge_index[0], pad_idx]).reshape(EG, GP)
    dst = jnp.concatenate([edge_index[1], pad_idx]).reshape(EG, GP)
    xp = jnp.zeros((NP, D), jnp.float32).at[:N].set(x)
    wcat = jnp.concatenate([W_mu, W_lv], axis=1)
    bcat = jnp.concatenate([b_mu, b_lv]).reshape(1, D)

    degp = _deg_kernel(dst)
    *g1q, dinv = _tc1(xp, W1, degp)
    s1 = _agg1(*g1q, src, dst)
    g2q = _tc2(s1, g1q, dinv, b1.reshape(1, D), wcat)
    s2 = _agg2(*g2q, src, dst)
    out = _tc3(s2, g2q, dinv, bcat)
    return out[:N, :64], out[:N, 64:]


# speedup vs baseline: 32.9259x; 1.2777x over previous
import functools

import jax
import jax.numpy as jnp
from jax import lax
from jax.experimental import pallas as pl
from jax.experimental.pallas import tpu as pltpu
from jax.experimental.pallas import tpu_sc as plsc

N = 10000
E = 320000
D = 128
DH = 64
DQ = 32
NC, NS = 2, 16
NW = NC * NS
GP = 128
NP = 10240
EG = 2560
GPW = EG // NW
RPT = NP // NS
DEG_W = 16
TBLK = 2048

_mesh = plsc.VectorSubcoreMesh(
    core_axis_name="c", subcore_axis_name="s", num_cores=NC, num_subcores=NS
)
_sc_params = pltpu.CompilerParams(use_tc_tiling_on_sc=False)


def _worker_ids():
    c = lax.axis_index("c")
    s = lax.axis_index("s")
    return c, s, c * NS + s


def _fill_zeros(zbuf):
    zeros = jnp.zeros((16,), jnp.float32)
    nvec = zbuf.shape[1] // 16

    def fill(i, _):
        zbuf[i // nvec, pl.ds((i % nvec) * 16, 16)] = zeros
        return 0

    lax.fori_loop(0, 128 * nvec, fill, 0)


def _zero_my_slice(zbuf, acc, s):
    for j in range(RPT // 128):
        pltpu.sync_copy(zbuf, acc.at[pl.ds(s * RPT + j * 128, 128)])


@functools.partial(
    pl.kernel,
    out_type=jax.ShapeDtypeStruct((NC, NP, DEG_W), jnp.float32),
    mesh=_mesh,
    scratch_types=[
        pltpu.VMEM((GPW, GP), jnp.int32),
        pltpu.VMEM((GP, DEG_W), jnp.float32),
        pltpu.VMEM((128, DEG_W), jnp.float32),
        pltpu.VMEM_SHARED((NP, DEG_W), jnp.float32),
    ],
    compiler_params=_sc_params,
)
def _deg_kernel(dst_hbm, out_hbm, idx_d, ones_v, zbuf, acc):
    c, s, w = _worker_ids()
    one = jnp.ones((16,), jnp.float32)

    def fill_ones(i, _):
        ones_v[i, pl.ds(0, 16)] = one
        return 0

    lax.fori_loop(0, GP, fill_ones, 0)
    _fill_zeros(zbuf)
    _zero_my_slice(zbuf, acc, s)
    pltpu.sync_copy(dst_hbm.at[pl.ds(w * GPW, GPW)], idx_d)
    plsc.subcore_barrier()

    def step(k, _):
        pltpu.sync_copy(ones_v, acc.at[idx_d.at[k]], add=True)
        return 0

    lax.fori_loop(0, GPW, step, 0)
    plsc.subcore_barrier()
    pltpu.sync_copy(
        acc.at[pl.ds(s * RPT, RPT)], out_hbm.at[c, pl.ds(s * RPT, RPT)]
    )


NBUF = 8
PD = NBUF // 2


def _make_agg():
    @functools.partial(
        pl.kernel,
        out_type=jax.ShapeDtypeStruct((NC, NP, D), jnp.float32),
        mesh=_mesh,
        scratch_types=[
            pltpu.VMEM((GPW, GP), jnp.int32),
            pltpu.VMEM((GPW, GP), jnp.int32),
            pltpu.VMEM((GPW, GP), jnp.int32),
            pltpu.VMEM((NBUF, GP, DQ), jnp.float32),
            pltpu.VMEM((128, DQ), jnp.float32),
            pltpu.VMEM_SHARED((NP, DQ), jnp.float32),
            pltpu.SemaphoreType.DMA((NBUF,)),
            pltpu.SemaphoreType.DMA((NBUF,)),
        ],
        compiler_params=_sc_params,
    )
    def agg(tbl, src4_hbm, dst_hbm, out_hbm,
            idx_s4, idx_q, idx_d, rows, zbuf, acc, gsem, ssem):
        c, s, w = _worker_ids()
        _fill_zeros(zbuf)
        pltpu.sync_copy(src4_hbm.at[pl.ds(w * GPW, GPW)], idx_s4)
        pltpu.sync_copy(dst_hbm.at[pl.ds(w * GPW, GPW)], idx_d)

        for sweep in range(4):
            idx = idx_s4 if sweep == 0 else idx_q
            if sweep > 0:
                def shift(i, _):
                    r, v = i // 8, (i % 8) * 16
                    idx_q[r, pl.ds(v, 16)] = idx_s4[r, pl.ds(v, 16)] + sweep
                    return 0

                lax.fori_loop(0, GPW * 8, shift, 0)
            _zero_my_slice(zbuf, acc, s)
            plsc.subcore_barrier()

            def start_gather(k, b):
                pltpu.async_copy(tbl.at[idx.at[k]], rows.at[b], gsem.at[b])

            def wait_gather(b):
                pltpu.make_async_copy(
                    tbl.at[idx.at[0]], rows.at[b], gsem.at[b]
                ).wait()

            def start_scatter(k, b):
                pltpu.async_copy(rows.at[b], acc.at[idx_d.at[k]], ssem.at[b],
                                 add=True)

            def wait_scatter(b):
                pltpu.make_async_copy(
                    rows.at[b], acc.at[idx_d.at[0]], ssem.at[b]
                ).wait()

            for b in range(PD):
                start_gather(b, b)

            def visit(k, b):
                wait_gather(b)
                start_scatter(k, b)

                @pl.when(k >= PD)
                def _():
                    wait_scatter((b + PD) % NBUF)

                @pl.when(k + PD < GPW)
                def _():
                    start_gather(k + PD, (b + PD) % NBUF)

            def macro(j, _):
                for b in range(NBUF):
                    visit(j * NBUF + b, b)
                return 0

            lax.fori_loop(0, GPW // NBUF, macro, 0)
            for b in range(PD):
                wait_scatter((GPW - PD + b) % NBUF)
            plsc.subcore_barrier()
            pltpu.sync_copy(
                acc.at[pl.ds(s * RPT, RPT)],
                out_hbm.at[c, pl.ds(s * RPT, RPT), pl.ds(sweep * DQ, DQ)],
            )
            plsc.subcore_barrier()

    return agg


_agg1 = _make_agg()
_agg2 = _make_agg()


def _tc1_body(x_ref, w1_ref, degp_ref, g1_ref, dinv_ref):
    deg = degp_ref[0][:, 0:1] + degp_ref[1][:, 0:1] + 1.0
    dinv = lax.rsqrt(deg)
    h = jnp.dot(x_ref[...], w1_ref[...], preferred_element_type=jnp.float32)
    g1_ref[...] = h * dinv
    dinv_ref[...] = dinv


def _tc2_body(s1_ref, g1_ref, dinv_ref, b1_ref, wcat_ref, g2_ref):
    dinv = dinv_ref[...]
    pre = (s1_ref[0] + s1_ref[1] + g1_ref[...]) * dinv + b1_ref[...]
    h = jnp.maximum(pre, 0.0)
    g2_ref[...] = (
        jnp.dot(h, wcat_ref[...], preferred_element_type=jnp.float32) * dinv
    )


def _tc3_body(s2_ref, g2_ref, dinv_ref, bcat_ref, out_ref):
    out_ref[...] = (
        (s2_ref[0] + s2_ref[1] + g2_ref[...]) * dinv_ref[...] + bcat_ref[...]
    )


_row_spec = pl.BlockSpec((TBLK, D), lambda i: (i, 0))
_col_spec = pl.BlockSpec((TBLK, 1), lambda i: (i, 0))
_par_spec = pl.BlockSpec((NC, TBLK, D), lambda i: (0, i, 0))
_w_spec = pl.BlockSpec((D, D), lambda i: (0, 0))
_b_spec = pl.BlockSpec((1, D), lambda i: (0, 0))
_GRID = (NP // TBLK,)


def _tc1(xp, W1, degp):
    return pl.pallas_call(
        _tc1_body,
        grid=_GRID,
        in_specs=[
            _row_spec,
            _w_spec,
            pl.BlockSpec((NC, TBLK, DEG_W), lambda i: (0, i, 0)),
        ],
        out_specs=[_row_spec, _col_spec],
        out_shape=[jax.ShapeDtypeStruct((NP, D), jnp.float32),
                   jax.ShapeDtypeStruct((NP, 1), jnp.float32)],
    )(xp, W1, degp)


def _tc2(s1, g1, dinv, b1, wcat):
    return pl.pallas_call(
        _tc2_body,
        grid=_GRID,
        in_specs=[_par_spec, _row_spec, _col_spec, _b_spec, _w_spec],
        out_specs=_row_spec,
        out_shape=jax.ShapeDtypeStruct((NP, D), jnp.float32),
    )(s1, g1, dinv, b1, wcat)


def _tc3(s2, g2, dinv, bcat):
    return pl.pallas_call(
        _tc3_body,
        grid=_GRID,
        in_specs=[_par_spec, _row_spec, _col_spec, _b_spec],
        out_specs=_row_spec,
        out_shape=jax.ShapeDtypeStruct((NP, D), jnp.float32),
    )(s2, g2, dinv, bcat)


def kernel(x, edge_index, W1, b1, W_mu, b_mu, W_lv, b_lv):
    pad_e = EG * GP - E
    pad_idx = N + jnp.arange(pad_e, dtype=jnp.int32) % (NP - N)
    src4 = (4 * jnp.concatenate([edge_index[0], pad_idx])).reshape(EG, GP)
    dst = jnp.concatenate([edge_index[1], pad_idx]).reshape(EG, GP)
    xp = jnp.zeros((NP, D), jnp.float32).at[:N].set(x)
    wcat = jnp.concatenate([W_mu, W_lv], axis=1)
    bcat = jnp.concatenate([b_mu, b_lv]).reshape(1, D)

    degp = _deg_kernel(dst)
    g1, dinv = _tc1(xp, W1, degp)
    s1 = _agg1(g1.reshape(4 * NP, DQ), src4, dst)
    g2 = _tc2(s1, g1, dinv, b1.reshape(1, D), wcat)
    s2 = _agg2(g2.reshape(4 * NP, DQ), src4, dst)
    out = _tc3(s2, g2, dinv, bcat)
    return out[:N, :64], out[:N, 64:]


# speedup vs baseline: 33.7981x; 1.0265x over previous
import functools

import jax
import jax.numpy as jnp
from jax import lax
from jax.experimental import pallas as pl
from jax.experimental.pallas import tpu as pltpu
from jax.experimental.pallas import tpu_sc as plsc

N = 10000
E = 320000
D = 128
DH = 64
DQ = 32
NC, NS = 2, 16
NW = NC * NS
GP = 128
NP = 10240
EG = 2560
GPW = EG // NW
EGR = E // GP
RPT = NP // NS
DEG_W = 16
TBLK = 2000

_mesh = plsc.VectorSubcoreMesh(
    core_axis_name="c", subcore_axis_name="s", num_cores=NC, num_subcores=NS
)
_sc_params = pltpu.CompilerParams(use_tc_tiling_on_sc=False)


def _worker_ids():
    c = lax.axis_index("c")
    s = lax.axis_index("s")
    return c, s, c * NS + s


def _fill_zeros(zbuf):
    zeros = jnp.zeros((16,), jnp.float32)
    nvec = zbuf.shape[1] // 16

    def fill(i, _):
        zbuf[i // nvec, pl.ds((i % nvec) * 16, 16)] = zeros
        return 0

    lax.fori_loop(0, 128 * nvec, fill, 0)


def _zero_my_slice(zbuf, acc, s):
    for j in range(RPT // 128):
        pltpu.sync_copy(zbuf, acc.at[pl.ds(s * RPT + j * 128, 128)])


def _load_group_indices(er_hbm, pad_hbm, row, idx_d, idx_s4=None):
    c = lax.axis_index("c")
    s = lax.axis_index("s")
    w = c * NS + s
    tail = EG - EGR
    real = GPW - tail

    @pl.when(w < NW - 1)
    def _():
        pltpu.sync_copy(er_hbm.at[row, pl.ds(w * GPW, GPW)], idx_d)
        if idx_s4 is not None:
            pltpu.sync_copy(er_hbm.at[0, pl.ds(w * GPW, GPW)], idx_s4)

    @pl.when(w == NW - 1)
    def _():
        pltpu.sync_copy(er_hbm.at[row, pl.ds(EGR - real, real)],
                        idx_d.at[pl.ds(0, real)])
        pltpu.sync_copy(pad_hbm, idx_d.at[pl.ds(real, tail)])
        if idx_s4 is not None:
            pltpu.sync_copy(er_hbm.at[0, pl.ds(EGR - real, real)],
                            idx_s4.at[pl.ds(0, real)])
            pltpu.sync_copy(pad_hbm, idx_s4.at[pl.ds(real, tail)])


@functools.partial(
    pl.kernel,
    out_type=jax.ShapeDtypeStruct((NC, NP, DEG_W), jnp.float32),
    mesh=_mesh,
    scratch_types=[
        pltpu.VMEM((GPW, GP), jnp.int32),
        pltpu.VMEM((GP, DEG_W), jnp.float32),
        pltpu.VMEM((128, DEG_W), jnp.float32),
        pltpu.VMEM_SHARED((NP, DEG_W), jnp.float32),
    ],
    compiler_params=_sc_params,
)
def _deg_kernel(er_hbm, pad_hbm, out_hbm, idx_d, ones_v, zbuf, acc):
    c, s, w = _worker_ids()
    one = jnp.ones((16,), jnp.float32)

    def fill_ones(i, _):
        ones_v[i, pl.ds(0, 16)] = one
        return 0

    lax.fori_loop(0, GP, fill_ones, 0)
    _fill_zeros(zbuf)
    _zero_my_slice(zbuf, acc, s)
    _load_group_indices(er_hbm, pad_hbm, 1, idx_d)
    plsc.subcore_barrier()

    def step(k, _):
        pltpu.sync_copy(ones_v, acc.at[idx_d.at[k]], add=True)
        return 0

    lax.fori_loop(0, GPW, step, 0)
    plsc.subcore_barrier()
    pltpu.sync_copy(
        acc.at[pl.ds(s * RPT, RPT)], out_hbm.at[c, pl.ds(s * RPT, RPT)]
    )


NBUF = 8
PD = NBUF // 2


def _make_agg():
    @functools.partial(
        pl.kernel,
        out_type=jax.ShapeDtypeStruct((NC, NP, D), jnp.float32),
        mesh=_mesh,
        scratch_types=[
            pltpu.VMEM((GPW, GP), jnp.int32),
            pltpu.VMEM((GPW, GP), jnp.int32),
            pltpu.VMEM((GPW, GP), jnp.int32),
            pltpu.VMEM((NBUF, GP, DQ), jnp.float32),
            pltpu.VMEM((128, DQ), jnp.float32),
            pltpu.VMEM_SHARED((NP, DQ), jnp.float32),
            pltpu.SemaphoreType.DMA((NBUF,)),
            pltpu.SemaphoreType.DMA((NBUF,)),
        ],
        compiler_params=_sc_params,
    )
    def agg(tbl, er_hbm, pad_hbm, out_hbm,
            idx_s4, idx_q, idx_d, rows, zbuf, acc, gsem, ssem):
        c, s, w = _worker_ids()
        _fill_zeros(zbuf)
        _load_group_indices(er_hbm, pad_hbm, 1, idx_d, idx_s4)

        def quadruple(i, _):
            r, v = i // 8, (i % 8) * 16
            idx_s4[r, pl.ds(v, 16)] = idx_s4[r, pl.ds(v, 16)] * 4
            return 0

        lax.fori_loop(0, GPW * 8, quadruple, 0)

        for sweep in range(4):
            idx = idx_s4 if sweep == 0 else idx_q
            if sweep > 0:
                def shift(i, _):
                    r, v = i // 8, (i % 8) * 16
                    idx_q[r, pl.ds(v, 16)] = idx_s4[r, pl.ds(v, 16)] + sweep
                    return 0

                lax.fori_loop(0, GPW * 8, shift, 0)
            _zero_my_slice(zbuf, acc, s)
            plsc.subcore_barrier()

            def start_gather(k, b):
                pltpu.async_copy(tbl.at[idx.at[k]], rows.at[b], gsem.at[b])

            def wait_gather(b):
                pltpu.make_async_copy(
                    tbl.at[idx.at[0]], rows.at[b], gsem.at[b]
                ).wait()

            def start_scatter(k, b):
                pltpu.async_copy(rows.at[b], acc.at[idx_d.at[k]], ssem.at[b],
                                 add=True)

            def wait_scatter(b):
                pltpu.make_async_copy(
                    rows.at[b], acc.at[idx_d.at[0]], ssem.at[b]
                ).wait()

            for b in range(PD):
                start_gather(b, b)

            def visit(k, b):
                wait_gather(b)
                start_scatter(k, b)

                @pl.when(k >= PD)
                def _():
                    wait_scatter((b + PD) % NBUF)

                @pl.when(k + PD < GPW)
                def _():
                    start_gather(k + PD, (b + PD) % NBUF)

            def macro(j, _):
                for b in range(NBUF):
                    visit(j * NBUF + b, b)
                return 0

            lax.fori_loop(0, GPW // NBUF, macro, 0)
            for b in range(PD):
                wait_scatter((GPW - PD + b) % NBUF)
            plsc.subcore_barrier()
            pltpu.sync_copy(
                acc.at[pl.ds(s * RPT, RPT)],
                out_hbm.at[c, pl.ds(s * RPT, RPT), pl.ds(sweep * DQ, DQ)],
            )
            plsc.subcore_barrier()

    return agg


_agg1 = _make_agg()
_agg2 = _make_agg()


def _tc1_body(x_ref, w1_ref, degp_ref, g1_ref, dinv_ref):
    deg = degp_ref[0][:, 0:1] + degp_ref[1][:, 0:1] + 1.0
    dinv = lax.rsqrt(deg)
    h = jnp.dot(x_ref[...], w1_ref[...], preferred_element_type=jnp.float32)
    g1_ref[...] = h * dinv
    dinv_ref[...] = dinv


def _tc2_body(s1_ref, g1_ref, dinv_ref, b1_ref, wcat_ref, g2_ref):
    dinv = dinv_ref[...]
    pre = (s1_ref[0] + s1_ref[1] + g1_ref[...]) * dinv + b1_ref[...]
    h = jnp.maximum(pre, 0.0)
    g2_ref[...] = (
        jnp.dot(h, wcat_ref[...], preferred_element_type=jnp.float32) * dinv
    )


def _tc3_body(s2_ref, g2_ref, dinv_ref, bcat_ref, mu_ref, lv_ref):
    out = (
        (s2_ref[0] + s2_ref[1] + g2_ref[...]) * dinv_ref[...] + bcat_ref[...]
    )
    mu_ref[...] = out[:, :DH]
    lv_ref[...] = out[:, DH:]


_row_spec = pl.BlockSpec((TBLK, D), lambda i: (i, 0))
_col_spec = pl.BlockSpec((TBLK, 1), lambda i: (i, 0))
_par_spec = pl.BlockSpec((NC, TBLK, D), lambda i: (0, i, 0))
_w_spec = pl.BlockSpec((D, D), lambda i: (0, 0))
_b_spec = pl.BlockSpec((1, D), lambda i: (0, 0))
_GRID = (N // TBLK,)


def _tc1(x, W1, degp):
    return pl.pallas_call(
        _tc1_body,
        grid=_GRID,
        in_specs=[
            _row_spec,
            _w_spec,
            pl.BlockSpec((NC, TBLK, DEG_W), lambda i: (0, i, 0)),
        ],
        out_specs=[_row_spec, _col_spec],
        out_shape=[jax.ShapeDtypeStruct((NP, D), jnp.float32),
                   jax.ShapeDtypeStruct((NP, 1), jnp.float32)],
    )(x, W1, degp)


def _tc2(s1, g1, dinv, b1, wcat):
    return pl.pallas_call(
        _tc2_body,
        grid=_GRID,
        in_specs=[_par_spec, _row_spec, _col_spec, _b_spec, _w_spec],
        out_specs=_row_spec,
        out_shape=jax.ShapeDtypeStruct((NP, D), jnp.float32),
    )(s1, g1, dinv, b1, wcat)


def _tc3(s2, g2, dinv, bcat):
    return pl.pallas_call(
        _tc3_body,
        grid=_GRID,
        in_specs=[_par_spec, _row_spec, _col_spec, _b_spec],
        out_specs=[pl.BlockSpec((TBLK, DH), lambda i: (i, 0))] * 2,
        out_shape=[jax.ShapeDtypeStruct((N, DH), jnp.float32)] * 2,
    )(s2, g2, dinv, bcat)


def kernel(x, edge_index, W1, b1, W_mu, b_mu, W_lv, b_lv):
    er = edge_index.reshape(2, EGR, GP)
    pad_idx = (N + jnp.arange((EG - EGR) * GP, dtype=jnp.int32)
               % (NP - N)).reshape(EG - EGR, GP)
    wcat = jnp.concatenate([W_mu, W_lv], axis=1)
    bcat = jnp.concatenate([b_mu, b_lv]).reshape(1, D)

    degp = _deg_kernel(er, pad_idx)
    g1, dinv = _tc1(x, W1, degp)
    s1 = _agg1(g1.reshape(4 * NP, DQ), er, pad_idx)
    g2 = _tc2(s1, g1, dinv, b1.reshape(1, D), wcat)
    s2 = _agg2(g2.reshape(4 * NP, DQ), er, pad_idx)
    mu, lv = _tc3(s2, g2, dinv, bcat)
    return mu, lv


# speedup vs baseline: 33.8662x; 1.0020x over previous
import functools

import jax
import jax.numpy as jnp
from jax import lax
from jax.experimental import pallas as pl
from jax.experimental.pallas import tpu as pltpu
from jax.experimental.pallas import tpu_sc as plsc

N = 10000
E = 320000
D = 128
DH = 64
DQ = 32
NC, NS = 2, 16
NW = NC * NS
GP = 128
NP = 10240
EG = 2560
GPW = EG // NW
EGR = E // GP
RPT = NP // NS
DEG_W = 16
TBLK = 2000

_mesh = plsc.VectorSubcoreMesh(
    core_axis_name="c", subcore_axis_name="s", num_cores=NC, num_subcores=NS
)
_sc_params = pltpu.CompilerParams(use_tc_tiling_on_sc=False)


def _worker_ids():
    c = lax.axis_index("c")
    s = lax.axis_index("s")
    return c, s, c * NS + s


def _fill_zeros(zbuf):
    zeros = jnp.zeros((16,), jnp.float32)
    nvec = zbuf.shape[1] // 16

    def fill(i, _):
        zbuf[i // nvec, pl.ds((i % nvec) * 16, 16)] = zeros
        return 0

    lax.fori_loop(0, 128 * nvec, fill, 0)


def _zero_my_slice(zbuf, acc, s):
    for j in range(RPT // 128):
        pltpu.sync_copy(zbuf, acc.at[pl.ds(s * RPT + j * 128, 128)])


def _load_group_indices(er_hbm, pad_hbm, row, idx_d, idx_s4=None):
    c = lax.axis_index("c")
    s = lax.axis_index("s")
    w = c * NS + s
    tail = EG - EGR
    real = GPW - tail

    @pl.when(w < NW - 1)
    def _():
        pltpu.sync_copy(er_hbm.at[row, pl.ds(w * GPW, GPW)], idx_d)
        if idx_s4 is not None:
            pltpu.sync_copy(er_hbm.at[0, pl.ds(w * GPW, GPW)], idx_s4)

    @pl.when(w == NW - 1)
    def _():
        pltpu.sync_copy(er_hbm.at[row, pl.ds(EGR - real, real)],
                        idx_d.at[pl.ds(0, real)])
        pltpu.sync_copy(pad_hbm, idx_d.at[pl.ds(real, tail)])
        if idx_s4 is not None:
            pltpu.sync_copy(er_hbm.at[0, pl.ds(EGR - real, real)],
                            idx_s4.at[pl.ds(0, real)])
            pltpu.sync_copy(pad_hbm, idx_s4.at[pl.ds(real, tail)])


@functools.partial(
    pl.kernel,
    out_type=jax.ShapeDtypeStruct((NC, NP, DEG_W), jnp.float32),
    mesh=_mesh,
    scratch_types=[
        pltpu.VMEM((GPW, GP), jnp.int32),
        pltpu.VMEM((GP, DEG_W), jnp.float32),
        pltpu.VMEM((128, DEG_W), jnp.float32),
        pltpu.VMEM_SHARED((NP, DEG_W), jnp.float32),
    ],
    compiler_params=_sc_params,
)
def _deg_kernel(er_hbm, pad_hbm, out_hbm, idx_d, ones_v, zbuf, acc):
    c, s, w = _worker_ids()
    one = jnp.ones((16,), jnp.float32)

    def fill_ones(i, _):
        ones_v[i, pl.ds(0, 16)] = one
        return 0

    lax.fori_loop(0, GP, fill_ones, 0)
    _fill_zeros(zbuf)
    _zero_my_slice(zbuf, acc, s)
    _load_group_indices(er_hbm, pad_hbm, 1, idx_d)
    plsc.subcore_barrier()

    def step(k, _):
        pltpu.sync_copy(ones_v, acc.at[idx_d.at[k]], add=True)
        return 0

    lax.fori_loop(0, GPW, step, 0)
    plsc.subcore_barrier()
    pltpu.sync_copy(
        acc.at[pl.ds(s * RPT, RPT)], out_hbm.at[c, pl.ds(s * RPT, RPT)]
    )


NBUF = 8
PD = NBUF // 2


def _make_agg():
    @functools.partial(
        pl.kernel,
        out_type=jax.ShapeDtypeStruct((NC, NP, D), jnp.float32),
        mesh=_mesh,
        scratch_types=[
            pltpu.VMEM((GPW, GP), jnp.int32),
            pltpu.VMEM((GPW, GP), jnp.int32),
            pltpu.VMEM((GPW, GP), jnp.int32),
            pltpu.VMEM((NBUF, GP, DQ), jnp.float32),
            pltpu.VMEM((128, DQ), jnp.float32),
            pltpu.VMEM_SHARED((NP, DQ), jnp.float32),
            pltpu.SemaphoreType.DMA((NBUF,)),
            pltpu.SemaphoreType.DMA((NBUF,)),
        ],
        compiler_params=_sc_params,
    )
    def agg(tbl, er_hbm, pad_hbm, out_hbm,
            idx_s4, idx_q, idx_d, rows, zbuf, acc, gsem, ssem):
        c, s, w = _worker_ids()
        _fill_zeros(zbuf)
        _load_group_indices(er_hbm, pad_hbm, 1, idx_d, idx_s4)

        def quadruple(i, _):
            r, v = i // 8, (i % 8) * 16
            idx_s4[r, pl.ds(v, 16)] = idx_s4[r, pl.ds(v, 16)] * 4
            return 0

        lax.fori_loop(0, GPW * 8, quadruple, 0)

        _zero_my_slice(zbuf, acc, s)
        for sweep in range(4):
            idx = idx_s4 if sweep == 0 else idx_q
            if sweep > 0:
                def shift(i, _):
                    r, v = i // 8, (i % 8) * 16
                    idx_q[r, pl.ds(v, 16)] = idx_s4[r, pl.ds(v, 16)] + sweep
                    return 0

                lax.fori_loop(0, GPW * 8, shift, 0)
            plsc.subcore_barrier()

            def start_gather(k, b):
                pltpu.async_copy(tbl.at[idx.at[k]], rows.at[b], gsem.at[b])

            def wait_gather(b):
                pltpu.make_async_copy(
                    tbl.at[idx.at[0]], rows.at[b], gsem.at[b]
                ).wait()

            def start_scatter(k, b):
                pltpu.async_copy(rows.at[b], acc.at[idx_d.at[k]], ssem.at[b],
                                 add=True)

            def wait_scatter(b):
                pltpu.make_async_copy(
                    rows.at[b], acc.at[idx_d.at[0]], ssem.at[b]
                ).wait()

            for b in range(PD):
                start_gather(b, b)

            def visit(k, b):
                wait_gather(b)
                start_scatter(k, b)

                @pl.when(k >= PD)
                def _():
                    wait_scatter((b + PD) % NBUF)

                @pl.when(k + PD < GPW)
                def _():
                    start_gather(k + PD, (b + PD) % NBUF)

            def macro(j, _):
                for b in range(NBUF):
                    visit(j * NBUF + b, b)
                return 0

            lax.fori_loop(0, GPW // NBUF, macro, 0)
            for b in range(PD):
                wait_scatter((GPW - PD + b) % NBUF)
            plsc.subcore_barrier()
            pltpu.sync_copy(
                acc.at[pl.ds(s * RPT, RPT)],
                out_hbm.at[c, pl.ds(s * RPT, RPT), pl.ds(sweep * DQ, DQ)],
            )
            if sweep < 3:
                _zero_my_slice(zbuf, acc, s)

    return agg


_agg1 = _make_agg()
_agg2 = _make_agg()


def _tc1_body(x_ref, w1_ref, degp_ref, g1_ref, dinv_ref):
    deg = degp_ref[0][:, 0:1] + degp_ref[1][:, 0:1] + 1.0
    dinv = lax.rsqrt(deg)
    h = jnp.dot(x_ref[...], w1_ref[...], preferred_element_type=jnp.float32)
    g1_ref[...] = h * dinv
    dinv_ref[...] = dinv


def _tc2_body(s1_ref, g1_ref, dinv_ref, b1_ref, wcat_ref, g2_ref):
    dinv = dinv_ref[...]
    pre = (s1_ref[0] + s1_ref[1] + g1_ref[...]) * dinv + b1_ref[...]
    h = jnp.maximum(pre, 0.0)
    g2_ref[...] = (
        jnp.dot(h, wcat_ref[...], preferred_element_type=jnp.float32) * dinv
    )


def _tc3_body(s2_ref, g2_ref, dinv_ref, bcat_ref, mu_ref, lv_ref):
    out = (
        (s2_ref[0] + s2_ref[1] + g2_ref[...]) * dinv_ref[...] + bcat_ref[...]
    )
    mu_ref[...] = out[:, :DH]
    lv_ref[...] = out[:, DH:]


_row_spec = pl.BlockSpec((TBLK, D), lambda i: (i, 0))
_col_spec = pl.BlockSpec((TBLK, 1), lambda i: (i, 0))
_par_spec = pl.BlockSpec((NC, TBLK, D), lambda i: (0, i, 0))
_w_spec = pl.BlockSpec((D, D), lambda i: (0, 0))
_b_spec = pl.BlockSpec((1, D), lambda i: (0, 0))
_GRID = (N // TBLK,)


def _tc1(x, W1, degp):
    return pl.pallas_call(
        _tc1_body,
        grid=_GRID,
        in_specs=[
            _row_spec,
            _w_spec,
            pl.BlockSpec((NC, TBLK, DEG_W), lambda i: (0, i, 0)),
        ],
        out_specs=[_row_spec, _col_spec],
        out_shape=[jax.ShapeDtypeStruct((NP, D), jnp.float32),
                   jax.ShapeDtypeStruct((NP, 1), jnp.float32)],
    )(x, W1, degp)


def _tc2(s1, g1, dinv, b1, wcat):
    return pl.pallas_call(
        _tc2_body,
        grid=_GRID,
        in_specs=[_par_spec, _row_spec, _col_spec, _b_spec, _w_spec],
        out_specs=_row_spec,
        out_shape=jax.ShapeDtypeStruct((NP, D), jnp.float32),
    )(s1, g1, dinv, b1, wcat)


def _tc3(s2, g2, dinv, bcat):
    return pl.pallas_call(
        _tc3_body,
        grid=_GRID,
        in_specs=[_par_spec, _row_spec, _col_spec, _b_spec],
        out_specs=[pl.BlockSpec((TBLK, DH), lambda i: (i, 0))] * 2,
        out_shape=[jax.ShapeDtypeStruct((N, DH), jnp.float32)] * 2,
    )(s2, g2, dinv, bcat)


def kernel(x, edge_index, W1, b1, W_mu, b_mu, W_lv, b_lv):
    er = edge_index.reshape(2, EGR, GP)
    pad_idx = (N + jnp.arange((EG - EGR) * GP, dtype=jnp.int32)
               % (NP - N)).reshape(EG - EGR, GP)
    wcat = jnp.concatenate([W_mu, W_lv], axis=1)
    bcat = jnp.concatenate([b_mu, b_lv]).reshape(1, D)

    degp = _deg_kernel(er, pad_idx)
    g1, dinv = _tc1(x, W1, degp)
    s1 = _agg1(g1.reshape(4 * NP, DQ), er, pad_idx)
    g2 = _tc2(s1, g1, dinv, b1.reshape(1, D), wcat)
    s2 = _agg2(g2.reshape(4 * NP, DQ), er, pad_idx)
    mu, lv = _tc3(s2, g2, dinv, bcat)
    return mu, lv
